# baseline (speedup 1.0000x reference)
import functools

import jax
import jax.numpy as jnp
from jax import lax
from jax.experimental import pallas as pl
from jax.experimental.pallas import tpu as pltpu
from jax.experimental.pallas import tpu_sc as plsc

_G = 4
_NB = 8
_VP = 32


def _sc_scatter_kernel(ei_hbm, a_hbm, ei_v, a_v):
    cid = lax.axis_index("c")
    sid = lax.axis_index("s")

    @pl.when(jnp.logical_and(cid == 0, sid == 0))
    def _():
        pltpu.sync_copy(ei_hbm, ei_v)
        zeros = jnp.zeros((16,), jnp.float32)
        nflat = a_v.shape[0]
        for k in range(nflat // 16):
            a_v[pl.ds(k * 16, 16)] = zeros
        ones = jnp.ones((16,), jnp.float32)
        E = ei_v.shape[0] // 2
        for j in range(E // 16):
            row = ei_v[pl.ds(j * 16, 16)]
            col = ei_v[pl.ds(E + j * 16, 16)]
            idx = col * _VP + row
            plsc.store_scatter(a_v, [idx], ones)
        pltpu.sync_copy(a_v, a_hbm)


def _build_k_kernel(ad_ref, k_ref):
    GV = k_ref.shape[0]
    V = GV // _G
    Aind = ad_ref[...][:, 0:V]
    deg = jnp.sum(Aind, axis=1, keepdims=True) + 1.0
    dinv = lax.rsqrt(deg)
    outer = lax.dot_general(dinv, dinv, (((1,), (1,)), ((), ())),
                            preferred_element_type=jnp.float32)
    eye = (lax.broadcasted_iota(jnp.int32, (V, V), 0)
           == lax.broadcasted_iota(jnp.int32, (V, V), 1)).astype(jnp.float32)
    A = (Aind + eye) * outer
    p = lax.broadcasted_iota(jnp.int32, (GV, GV), 0)
    q = lax.broadcasted_iota(jnp.int32, (GV, GV), 1)
    same_block = ((p // V) == (q // V)).astype(jnp.float32)
    Pv = (lax.broadcasted_iota(jnp.int32, (GV, V), 0) % V
          == lax.broadcasted_iota(jnp.int32, (GV, V), 1)).astype(jnp.float32)
    t1 = lax.dot_general(Pv, A, (((1,), (1,)), ((), ())),
                         preferred_element_type=jnp.float32)
    at_big = lax.dot_general(t1, Pv, (((1,), (1,)), ((), ())),
                             preferred_element_type=jnp.float32)
    k_ref[...] = at_big * same_block


def _main_kernel(x_ref, w_ref, k_ref, b_ref, o_ref):
    NB, O, TG, GV = o_ref.shape
    K = k_ref[...]
    for b in range(NB):
        y = lax.dot_general(w_ref[...], x_ref[b], (((0,), (0,)), ((), ())),
                            preferred_element_type=jnp.float32)
        y3 = y.reshape(O, TG, GV) + jnp.zeros((1, 1, GV), jnp.float32)
        y2 = y3.reshape(O * TG, GV)
        u = lax.dot_general(y2, K, (((1,), (0,)), ((), ())),
                            preferred_element_type=jnp.float32)
        o_ref[b] = u.reshape(O, TG, GV) + b_ref[...].reshape(O, 1, 1)


def kernel(x, W, b, edge_index):
    n, c, t, v = x.shape
    o = W.shape[1]
    e = edge_index.shape[1]
    ei = edge_index.astype(jnp.int32)
    gv = _G * v
    tg = t // _G

    sc_scatter = functools.partial(
        pl.kernel,
        out_type=jax.ShapeDtypeStruct((v * _VP,), jnp.float32),
        mesh=plsc.VectorSubcoreMesh(core_axis_name="c", subcore_axis_name="s"),
        compiler_params=pltpu.CompilerParams(needs_layout_passes=False),
        scratch_types=[
            pltpu.VMEM((2 * e,), jnp.int32),
            pltpu.VMEM((v * _VP,), jnp.float32),
        ],
    )(_sc_scatter_kernel)
    a_flat = sc_scatter(ei.reshape(2 * e))
    ad = a_flat.reshape(v, _VP)

    K = pl.pallas_call(
        _build_k_kernel,
        out_shape=jax.ShapeDtypeStruct((gv, gv), jnp.float32),
    )(ad)

    b2 = b.reshape(o, 1)
    x2 = x.reshape(n, c, t * v)

    out = pl.pallas_call(
        _main_kernel,
        grid=(n // _NB,),
        in_specs=[
            pl.BlockSpec((_NB, c, t * v), lambda i: (i, 0, 0)),
            pl.BlockSpec((c, o), lambda i: (0, 0)),
            pl.BlockSpec((gv, gv), lambda i: (0, 0)),
            pl.BlockSpec((o, 1), lambda i: (0, 0)),
        ],
        out_specs=pl.BlockSpec((_NB, o, tg, gv), lambda i: (i, 0, 0, 0)),
        out_shape=jax.ShapeDtypeStruct((n, o, tg, gv), jnp.float32),
    )(x2, W, K, b2)
    return out.reshape(n, o, t, v)
